# baseline (speedup 1.0000x reference)
import functools

import jax
import jax.numpy as jnp
from jax import lax
from jax.experimental import pallas as pl

E = 64
TOPK = 2
D = 768
DFF = 1536
DE = DFF // E
TB = 256


def _body(x_ref, cen_ref, wr_ref, w1_ref, b1_ref, w2_ref, b2_ref, o_ref):
    x = x_ref[...]
    cen = cen_ref[...]
    wr = wr_ref[...]

    cn = cen / (jnp.sqrt(jnp.sum(cen * cen, axis=1, keepdims=True)) + 1e-12)
    xn = x / (jnp.sqrt(jnp.sum(x * x, axis=1, keepdims=True)) + 1e-12)
    nt = (((1,), (1,)), ((), ()))
    cos = lax.dot_general(xn, cn, nt, preferred_element_type=jnp.float32,
                          precision=lax.Precision.HIGHEST)
    route = lax.dot_general(x, wr, nt, preferred_element_type=jnp.float32,
                            precision=lax.Precision.HIGHEST)
    logits = cos + route

    ie = lax.broadcasted_iota(jnp.int32, (TB, E), 1)
    m1 = jnp.max(logits, axis=1, keepdims=True)
    i1 = jnp.min(jnp.where(logits >= m1, ie, E), axis=1, keepdims=True)
    l2 = jnp.where(ie == i1, -jnp.inf, logits)
    m2 = jnp.max(l2, axis=1, keepdims=True)
    i2 = jnp.min(jnp.where(l2 >= m2, ie, E), axis=1, keepdims=True)
    d = jnp.exp(m2 - m1)
    g1 = 1.0 / (1.0 + d)
    g2 = d * g1
    gate = jnp.where(ie == i1, g1, 0.0) + jnp.where(ie == i2, g2, 0.0)

    h = lax.dot_general(x, w1_ref[...], nt, preferred_element_type=jnp.float32)
    h = h + b1_ref[...]
    h = 0.5 * h * (1.0 + lax.erf(h * 0.7071067811865476))

    ce = lax.broadcasted_iota(jnp.int32, (E, DFF), 1)
    re = lax.broadcasted_iota(jnp.int32, (E, DFF), 0) * DE
    R = ((ce >= re) & (ce < re + DE)).astype(jnp.float32)
    gw = lax.dot_general(gate, R, (((1,), (0,)), ((), ())),
                         preferred_element_type=jnp.float32)

    out = lax.dot_general(h * gw, w2_ref[...], (((1,), (0,)), ((), ())),
                          preferred_element_type=jnp.float32)
    out = out + lax.dot_general(gate, b2_ref[...], (((1,), (0,)), ((), ())),
                                preferred_element_type=jnp.float32)
    o_ref[...] = out


def kernel(x, centroids, Wr, w1, b1, w2, b2):
    B, S, _ = x.shape
    N = B * S
    xf = x.reshape(N, D)
    w1f = w1.reshape(DFF, D)
    w2f = jnp.transpose(w2, (0, 2, 1)).reshape(DFF, D)
    b1f = b1.reshape(1, DFF)

    grid = N // TB
    full = lambda *_: (0, 0)
    out = pl.pallas_call(
        _body,
        grid=(grid,),
        in_specs=[
            pl.BlockSpec((TB, D), lambda i: (i, 0)),
            pl.BlockSpec((E, D), full),
            pl.BlockSpec((E, D), full),
            pl.BlockSpec((DFF, D), full),
            pl.BlockSpec((1, DFF), full),
            pl.BlockSpec((DFF, D), full),
            pl.BlockSpec((E, D), full),
        ],
        out_specs=pl.BlockSpec((TB, D), lambda i: (i, 0)),
        out_shape=jax.ShapeDtypeStruct((N, D), jnp.float32),
    )(xf, centroids, Wr, w1f, b1f, w2f, b2)
    return out.reshape(B, S, D)


# speedup vs baseline: 2.1850x; 2.1850x over previous
import functools

import jax
import jax.numpy as jnp
from jax import lax
from jax.experimental import pallas as pl

E = 64
TOPK = 2
D = 768
DFF = 1536
DE = DFF // E
TB = 256


def _body(x_ref, cen_ref, wr_ref, w1_ref, b1_ref, w2_ref, b2_ref, o_ref):
    x = x_ref[...]
    cen = cen_ref[...]
    wr = wr_ref[...]

    cn = cen / (jnp.sqrt(jnp.sum(cen * cen, axis=1, keepdims=True)) + 1e-12)
    xn = x / (jnp.sqrt(jnp.sum(x * x, axis=1, keepdims=True)) + 1e-12)
    nt = (((1,), (1,)), ((), ()))
    cos = lax.dot_general(xn, cn, nt, preferred_element_type=jnp.float32)
    route = lax.dot_general(x, wr, nt, preferred_element_type=jnp.float32)
    logits = cos + route

    ie = lax.broadcasted_iota(jnp.int32, (TB, E), 1)
    m1 = jnp.max(logits, axis=1, keepdims=True)
    i1 = jnp.min(jnp.where(logits >= m1, ie, E), axis=1, keepdims=True)
    l2 = jnp.where(ie == i1, -jnp.inf, logits)
    m2 = jnp.max(l2, axis=1, keepdims=True)
    i2 = jnp.min(jnp.where(l2 >= m2, ie, E), axis=1, keepdims=True)
    d = jnp.exp(m2 - m1)
    g1 = 1.0 / (1.0 + d)
    g2 = d * g1
    gate = jnp.where(ie == i1, g1, 0.0) + jnp.where(ie == i2, g2, 0.0)

    h = lax.dot_general(x.astype(jnp.bfloat16), w1_ref[...].astype(jnp.bfloat16),
                        nt, preferred_element_type=jnp.float32)
    h = h + b1_ref[...]
    h = 0.5 * h * (1.0 + lax.erf(h * 0.7071067811865476))

    ce = lax.broadcasted_iota(jnp.int32, (E, DFF), 1)
    re = lax.broadcasted_iota(jnp.int32, (E, DFF), 0) * DE
    R = ((ce >= re) & (ce < re + DE)).astype(jnp.float32)
    gw = lax.dot_general(gate, R, (((1,), (0,)), ((), ())),
                         preferred_element_type=jnp.float32)

    out = lax.dot_general((h * gw).astype(jnp.bfloat16),
                          w2_ref[...].astype(jnp.bfloat16), (((1,), (0,)), ((), ())),
                          preferred_element_type=jnp.float32)
    out = out + lax.dot_general(gate, b2_ref[...], (((1,), (0,)), ((), ())),
                                preferred_element_type=jnp.float32)
    o_ref[...] = out


def kernel(x, centroids, Wr, w1, b1, w2, b2):
    B, S, _ = x.shape
    N = B * S
    xf = x.reshape(N, D)
    w1f = w1.reshape(DFF, D)
    w2f = jnp.transpose(w2, (0, 2, 1)).reshape(DFF, D)
    b1f = b1.reshape(1, DFF)

    grid = N // TB
    full = lambda *_: (0, 0)
    out = pl.pallas_call(
        _body,
        grid=(grid,),
        in_specs=[
            pl.BlockSpec((TB, D), lambda i: (i, 0)),
            pl.BlockSpec((E, D), full),
            pl.BlockSpec((E, D), full),
            pl.BlockSpec((DFF, D), full),
            pl.BlockSpec((1, DFF), full),
            pl.BlockSpec((DFF, D), full),
            pl.BlockSpec((E, D), full),
        ],
        out_specs=pl.BlockSpec((TB, D), lambda i: (i, 0)),
        out_shape=jax.ShapeDtypeStruct((N, D), jnp.float32),
    )(xf, centroids, Wr, w1f, b1f, w2f, b2)
    return out.reshape(B, S, D)


# speedup vs baseline: 2.3772x; 1.0879x over previous
import functools

import jax
import jax.numpy as jnp
from jax import lax
from jax.experimental import pallas as pl

E = 64
TOPK = 2
D = 768
DFF = 1536
DE = DFF // E
TB = 1024


def _body(x_ref, cen_ref, wr_ref, w1_ref, b1_ref, w2_ref, b2_ref, o_ref):
    x = x_ref[...]
    cen = cen_ref[...]
    wr = wr_ref[...]

    cn = cen / (jnp.sqrt(jnp.sum(cen * cen, axis=1, keepdims=True)) + 1e-12)
    xn = x / (jnp.sqrt(jnp.sum(x * x, axis=1, keepdims=True)) + 1e-12)
    nt = (((1,), (1,)), ((), ()))
    cos = lax.dot_general(xn, cn, nt, preferred_element_type=jnp.float32)
    route = lax.dot_general(x, wr, nt, preferred_element_type=jnp.float32)
    logits = cos + route

    ie = lax.broadcasted_iota(jnp.int32, (TB, E), 1)
    m1 = jnp.max(logits, axis=1, keepdims=True)
    i1 = jnp.min(jnp.where(logits >= m1, ie, E), axis=1, keepdims=True)
    l2 = jnp.where(ie == i1, -jnp.inf, logits)
    m2 = jnp.max(l2, axis=1, keepdims=True)
    i2 = jnp.min(jnp.where(l2 >= m2, ie, E), axis=1, keepdims=True)
    d = jnp.exp(m2 - m1)
    g1 = 1.0 / (1.0 + d)
    g2 = d * g1
    gate = jnp.where(ie == i1, g1, 0.0) + jnp.where(ie == i2, g2, 0.0)

    h = lax.dot_general(x.astype(jnp.bfloat16), w1_ref[...],
                        nt, preferred_element_type=jnp.float32)
    h = h + b1_ref[...]
    h = 0.5 * h * (1.0 + lax.erf(h * 0.7071067811865476))

    ce = lax.broadcasted_iota(jnp.int32, (E, DFF), 1)
    re = lax.broadcasted_iota(jnp.int32, (E, DFF), 0) * DE
    R = ((ce >= re) & (ce < re + DE)).astype(jnp.float32)
    gw = lax.dot_general(gate, R, (((1,), (0,)), ((), ())),
                         preferred_element_type=jnp.float32)

    out = lax.dot_general((h * gw).astype(jnp.bfloat16),
                          w2_ref[...], (((1,), (0,)), ((), ())),
                          preferred_element_type=jnp.float32)
    out = out + lax.dot_general(gate, b2_ref[...], (((1,), (0,)), ((), ())),
                                preferred_element_type=jnp.float32)
    o_ref[...] = out


def kernel(x, centroids, Wr, w1, b1, w2, b2):
    B, S, _ = x.shape
    N = B * S
    xf = x.reshape(N, D)
    w1f = w1.reshape(DFF, D).astype(jnp.bfloat16)
    w2f = jnp.transpose(w2, (0, 2, 1)).reshape(DFF, D).astype(jnp.bfloat16)
    b1f = b1.reshape(1, DFF)

    grid = N // TB
    full = lambda *_: (0, 0)
    out = pl.pallas_call(
        _body,
        grid=(grid,),
        in_specs=[
            pl.BlockSpec((TB, D), lambda i: (i, 0)),
            pl.BlockSpec((E, D), full),
            pl.BlockSpec((E, D), full),
            pl.BlockSpec((DFF, D), full),
            pl.BlockSpec((1, DFF), full),
            pl.BlockSpec((DFF, D), full),
            pl.BlockSpec((E, D), full),
        ],
        out_specs=pl.BlockSpec((TB, D), lambda i: (i, 0)),
        out_shape=jax.ShapeDtypeStruct((N, D), jnp.float32),
    )(xf, centroids, Wr, w1f, b1f, w2f, b2)
    return out.reshape(B, S, D)
